# baseline (speedup 1.0000x reference)
import jax
import jax.numpy as jnp
from jax.experimental import pallas as pl
from jax.experimental.pallas import tpu as pltpu

B = 8
L = 2048
D = 1024
S = 128
C = L // S
BB = 8


def _body(hc_ref, wq_ref, wk_ref, o_ref,
          carry_ref, qprev_ref, qqprev_ref, a_ref, pb_ref, hp_ref):
    c = pl.program_id(1)

    a_prev = a_ref[...]
    pb_prev = pb_ref[...]

    @pl.when(c > 0)
    def _scan_phase():
        hp = hp_ref[...]

        rows = jax.lax.broadcasted_iota(jnp.int32, (BB, S, S), 1)
        cols = jax.lax.broadcasted_iota(jnp.int32, (BB, S, S), 2)
        w = jnp.where(rows == cols, pb_prev, 0.0)
        a = a_prev
        stride = 1
        while stride < S:
            a_sh = jnp.concatenate(
                [jnp.ones((BB, stride, 1), jnp.float32), a[:, :-stride]],
                axis=1)
            w_sh = jnp.concatenate(
                [jnp.zeros((BB, stride, S), jnp.float32), w[:, :-stride]],
                axis=1)
            w = w + a * w_sh
            a = a * a_sh
            stride *= 2
        wi = w + jnp.where(rows == cols, 1.0, 0.0)
        full = jax.lax.dot_general(
            wi, hp, (((2,), (1,)), ((0,), (0,))),
            precision=jax.lax.Precision.DEFAULT,
            preferred_element_type=jnp.float32)

        @pl.when(c == 1)
        def _zero_carry():
            carry_ref[...] = jnp.zeros_like(carry_ref)

        carry = carry_ref[...]
        out = full + a * carry
        o_ref[...] = out
        carry_ref[...] = out[:, -1:] - hp[:, -1:]

    @pl.when(c < C)
    def _router_phase():
        h = hc_ref[...]
        hf = h.reshape((BB * S, D)).astype(jnp.bfloat16)
        dn = (((1,), (1,)), ((), ()))
        q = jax.lax.dot_general(hf, wq_ref[...], dn,
                                precision=jax.lax.Precision.DEFAULT,
                                preferred_element_type=jnp.float32)
        k = jax.lax.dot_general(hf, wk_ref[...], dn,
                                precision=jax.lax.Precision.DEFAULT,
                                preferred_element_type=jnp.float32)
        q = q.reshape((BB, S, D))
        k = k.reshape((BB, S, D))
        qq = jnp.sum(q * q, axis=2, keepdims=True)
        kk = jnp.sum(k * k, axis=2, keepdims=True)
        q_sh = jnp.concatenate([qprev_ref[...], q[:, :-1]], axis=1)
        qq_sh = jnp.concatenate([qqprev_ref[...], qq[:, :-1]], axis=1)
        cross = jnp.sum(q_sh * k, axis=2, keepdims=True)
        eps = 1e-12
        denom = (jnp.maximum(jnp.sqrt(qq_sh), eps)
                 * jnp.maximum(jnp.sqrt(kk), eps))
        cos = cross / denom
        prob = jnp.clip((1.0 - cos) * 0.5, 0.0, 1.0)
        row = jax.lax.broadcasted_iota(jnp.int32, (BB, S, 1), 1)
        prob = jnp.where(jnp.logical_and(c == 0, row == 0), 1.0, prob)
        mask = prob > 0.5
        a_ref[...] = jnp.where(mask, 1.0 - prob, 1.0)
        pb_ref[...] = jnp.where(mask, prob, 0.0)
        qprev_ref[...] = q[:, -1:]
        qqprev_ref[...] = qq[:, -1:]
        hp_ref[...] = h


@jax.jit
def kernel(hidden_states, state, Wq, Wk):
    del state
    grid = (B // BB, C + 1)
    out = pl.pallas_call(
        _body,
        grid=grid,
        in_specs=[
            pl.BlockSpec((BB, S, D),
                         lambda b, c: (b, jnp.minimum(c, C - 1), 0)),
            pl.BlockSpec((D, D), lambda b, c: (0, 0)),
            pl.BlockSpec((D, D), lambda b, c: (0, 0)),
        ],
        out_specs=pl.BlockSpec(
            (BB, S, D), lambda b, c: (b, jnp.maximum(c - 1, 0), 0)),
        out_shape=jax.ShapeDtypeStruct((B, L, D), jnp.float32),
        scratch_shapes=[
            pltpu.VMEM((BB, 1, D), jnp.float32),
            pltpu.VMEM((BB, 1, D), jnp.float32),
            pltpu.VMEM((BB, 1, 1), jnp.float32),
            pltpu.VMEM((BB, S, 1), jnp.float32),
            pltpu.VMEM((BB, S, 1), jnp.float32),
            pltpu.VMEM((BB, S, D), jnp.float32),
        ],
    )(hidden_states, Wq, Wk)
    return out


# speedup vs baseline: 1.0036x; 1.0036x over previous
import jax
import jax.numpy as jnp
from jax.experimental import pallas as pl
from jax.experimental.pallas import tpu as pltpu

B = 8
L = 2048
D = 1024
S = 128
C = L // S
BB = 8


def _body(hc_ref, wq_ref, wk_ref, o_ref,
          carry_ref, qprev_ref, qqprev_ref, a_ref, pb_ref, hp_ref):
    c = pl.program_id(1)

    a_prev = a_ref[...]
    pb_prev = pb_ref[...]

    @pl.when(c > 0)
    def _scan_phase():
        hp = hp_ref[...]

        rows = jax.lax.broadcasted_iota(jnp.int32, (BB, S, S), 1)
        cols = jax.lax.broadcasted_iota(jnp.int32, (BB, S, S), 2)
        w = jnp.where(rows == cols, pb_prev, 0.0)
        a = a_prev
        stride = 1
        while stride < S:
            a_sh = jnp.concatenate(
                [jnp.ones((BB, stride, 1), jnp.float32), a[:, :-stride]],
                axis=1)
            w_sh = jnp.concatenate(
                [jnp.zeros((BB, stride, S), jnp.float32), w[:, :-stride]],
                axis=1)
            w = w + a * w_sh
            a = a * a_sh
            stride *= 2
        wi = w + jnp.where(rows == cols, 1.0, 0.0)
        full = jax.lax.dot_general(
            wi, hp, (((2,), (1,)), ((0,), (0,))),
            precision=jax.lax.Precision.DEFAULT,
            preferred_element_type=jnp.float32)

        @pl.when(c == 1)
        def _zero_carry():
            carry_ref[...] = jnp.zeros_like(carry_ref)

        carry = carry_ref[...]
        out = full + a * carry
        o_ref[...] = out
        carry_ref[...] = out[:, -1:] - hp[:, -1:]

    @pl.when(c < C)
    def _router_phase():
        h = hc_ref[...]
        hf = h.reshape((BB * S, D))
        dn = (((1,), (1,)), ((), ()))
        q = jax.lax.dot_general(hf, wq_ref[...], dn,
                                precision=jax.lax.Precision.DEFAULT,
                                preferred_element_type=jnp.float32)
        k = jax.lax.dot_general(hf, wk_ref[...], dn,
                                precision=jax.lax.Precision.DEFAULT,
                                preferred_element_type=jnp.float32)
        q = q.reshape((BB, S, D))
        k = k.reshape((BB, S, D))
        qq = jnp.sum(q * q, axis=2, keepdims=True)
        kk = jnp.sum(k * k, axis=2, keepdims=True)
        q_sh = jnp.concatenate([qprev_ref[...], q[:, :-1]], axis=1)
        qq_sh = jnp.concatenate([qqprev_ref[...], qq[:, :-1]], axis=1)
        cross = jnp.sum(q_sh * k, axis=2, keepdims=True)
        eps = 1e-12
        denom = (jnp.maximum(jnp.sqrt(qq_sh), eps)
                 * jnp.maximum(jnp.sqrt(kk), eps))
        cos = cross / denom
        prob = jnp.clip((1.0 - cos) * 0.5, 0.0, 1.0)
        row = jax.lax.broadcasted_iota(jnp.int32, (BB, S, 1), 1)
        prob = jnp.where(jnp.logical_and(c == 0, row == 0), 1.0, prob)
        mask = prob > 0.5
        a_ref[...] = jnp.where(mask, 1.0 - prob, 1.0)
        pb_ref[...] = jnp.where(mask, prob, 0.0)
        qprev_ref[...] = q[:, -1:]
        qqprev_ref[...] = qq[:, -1:]
        hp_ref[...] = h


@jax.jit
def kernel(hidden_states, state, Wq, Wk):
    del state
    grid = (B // BB, C + 1)
    out = pl.pallas_call(
        _body,
        grid=grid,
        in_specs=[
            pl.BlockSpec((BB, S, D),
                         lambda b, c: (b, jnp.minimum(c, C - 1), 0)),
            pl.BlockSpec((D, D), lambda b, c: (0, 0)),
            pl.BlockSpec((D, D), lambda b, c: (0, 0)),
        ],
        out_specs=pl.BlockSpec(
            (BB, S, D), lambda b, c: (b, jnp.maximum(c - 1, 0), 0)),
        out_shape=jax.ShapeDtypeStruct((B, L, D), jnp.float32),
        scratch_shapes=[
            pltpu.VMEM((BB, 1, D), jnp.float32),
            pltpu.VMEM((BB, 1, D), jnp.float32),
            pltpu.VMEM((BB, 1, 1), jnp.float32),
            pltpu.VMEM((BB, S, 1), jnp.float32),
            pltpu.VMEM((BB, S, 1), jnp.float32),
            pltpu.VMEM((BB, S, D), jnp.float32),
        ],
    )(hidden_states, Wq, Wk)
    return out
